# Initial kernel scaffold; baseline (speedup 1.0000x reference)
import jax
import jax.numpy as jnp
from jax.experimental import pallas as pl


def kernel(program_ids, pitch_tokens, velocity_tokens, note_durations_beats, attention_mask, gm_W, dr_W, vel_W, inst_W, W1, b1, W2, b2):
    raise NotImplementedError("write your pallas kernel here")


# speedup vs baseline: 3.7680x; 3.7680x over previous
import jax
import jax.numpy as jnp
from jax import lax
from jax.experimental import pallas as pl
from jax.experimental.pallas import tpu as pltpu
from jax.experimental.pallas import tpu_sc as plsc

B, P, T, D = 16, 8, 2048, 128
N = B * P * T
DRUMS_ID = 128
TRACKS = B * P
NC, NS = 2, 16
NW = NC * NS
TPW = TRACKS // NW
CHUNK = 256
NCH = T // CHUNK


def _sc_embed_body(gmdr, velW, instW, pitch, vel, off16, pids, e_out,
                   pidx_v, vidx_v, acc_v, velrows_v, instrows_v, off_v,
                   pid_v, sem1, sem2):
    wid = lax.axis_index("s") * NC + lax.axis_index("c")
    pltpu.sync_copy(pids, pid_v)
    pltpu.sync_copy(off16, off_v)
    pltpu.async_copy(instW.at[pid_v], instrows_v, sem1).wait()
    for trk in range(TPW):
        track = wid * TPW + trk
        off_vec = off_v[pl.ds(track * 16, 16)]
        inst_vecs = [instrows_v[track, pl.ds(w * 16, 16)] for w in range(8)]
        for ch in range(NCH):
            base = track * T + ch * CHUNK
            pltpu.sync_copy(pitch.at[pl.ds(base, CHUNK)], pidx_v)
            pltpu.sync_copy(vel.at[pl.ds(base, CHUNK)], vidx_v)
            for i in range(CHUNK // 16):
                s = pl.ds(i * 16, 16)
                pidx_v[s] = pidx_v[s] + off_vec
            cp1 = pltpu.async_copy(gmdr.at[pidx_v], acc_v, sem1)
            cp2 = pltpu.async_copy(velW.at[vidx_v], velrows_v, sem2)
            cp1.wait()
            cp2.wait()

            def addbody(r, carry):
                for w in range(8):
                    s = pl.ds(w * 16, 16)
                    acc_v[r, s] = acc_v[r, s] + velrows_v[r, s] + inst_vecs[w]
                return carry

            lax.fori_loop(0, CHUNK, addbody, 0)
            pltpu.sync_copy(acc_v, e_out.at[pl.ds(base, CHUNK)])


def _make_sc_embed():
    return pl.kernel(
        _sc_embed_body,
        out_type=jax.ShapeDtypeStruct((N, D), jnp.float32),
        mesh=plsc.VectorSubcoreMesh(core_axis_name="c", subcore_axis_name="s"),
        scratch_types=[
            pltpu.VMEM((CHUNK,), jnp.int32),
            pltpu.VMEM((CHUNK,), jnp.int32),
            pltpu.VMEM((CHUNK, D), jnp.float32),
            pltpu.VMEM((CHUNK, D), jnp.float32),
            pltpu.VMEM((TRACKS, D), jnp.float32),
            pltpu.VMEM((TRACKS * 16,), jnp.int32),
            pltpu.VMEM((TRACKS,), jnp.int32),
            pltpu.SemaphoreType.DMA,
            pltpu.SemaphoreType.DMA,
        ],
    )


BLK = 1024


def _tc_body(e_ref, dur_ref, mask_ref, w1_ref, b1_ref, w2_ref, b2_ref,
             out_ref):
    d = jnp.log(1.0 + dur_ref[...])
    h = d * w1_ref[...] + b1_ref[...]
    h = h * (1.0 / (1.0 + jnp.exp(-h)))
    h2 = lax.dot_general(h, w2_ref[...], (((1,), (1,)), ((), ())),
                         preferred_element_type=jnp.float32)
    out_ref[...] = (e_ref[...] + h2 + b2_ref[...]) * mask_ref[...]


def kernel(program_ids, pitch_tokens, velocity_tokens, note_durations_beats,
           attention_mask, gm_W, dr_W, vel_W, inst_W, W1, b1, W2, b2):
    pids = program_ids.reshape(-1).astype(jnp.int32)
    gmdr = jnp.concatenate([gm_W, dr_W], axis=0)
    off = jnp.where(pids == DRUMS_ID, 512, 0).astype(jnp.int32)
    off16 = jnp.broadcast_to(off[:, None], (TRACKS, 16)).reshape(-1)
    pitch = pitch_tokens.reshape(-1).astype(jnp.int32)
    vel = velocity_tokens.reshape(-1).astype(jnp.int32)

    e2d = _make_sc_embed()(gmdr, vel_W, inst_W, pitch, vel, off16, pids)

    dur2d = note_durations_beats.reshape(N, 1)
    mask2d = attention_mask.reshape(N, 1).astype(jnp.float32)
    w1row = W1.reshape(1, D)
    b1row = b1.reshape(1, D)
    b2row = b2.reshape(1, D)

    out2d = pl.pallas_call(
        _tc_body,
        grid=(N // BLK,),
        in_specs=[
            pl.BlockSpec((BLK, D), lambda i: (i, 0)),
            pl.BlockSpec((BLK, 1), lambda i: (i, 0)),
            pl.BlockSpec((BLK, 1), lambda i: (i, 0)),
            pl.BlockSpec((1, D), lambda i: (0, 0)),
            pl.BlockSpec((1, D), lambda i: (0, 0)),
            pl.BlockSpec((D, D), lambda i: (0, 0)),
            pl.BlockSpec((1, D), lambda i: (0, 0)),
        ],
        out_specs=pl.BlockSpec((BLK, D), lambda i: (i, 0)),
        out_shape=jax.ShapeDtypeStruct((N, D), jnp.float32),
    )(e2d, dur2d, mask2d, w1row, b1row, W2, b2row)

    return out2d.reshape(B, P, T, D)
